# baseline (speedup 1.0000x reference)
import functools

import jax
import jax.numpy as jnp
from jax import lax
from jax.experimental import pallas as pl
from jax.experimental.pallas import tpu as pltpu
from jax.experimental.pallas import tpu_sc as plsc

M = 1000000
D = 32
B = 819200

NC = 2
NS = 16
MC = 25000
NCHUNK = M // MC
CPC = NCHUNK // NC
SLICE = B // NS
NVEC = SLICE // 16
ROWS_PER_TILE = 1568
LAST_OFF = MC - ROWS_PER_TILE
TRASH = 16
PCAP = 2560


def _body(mem_hbm, idx_hbm, val_hbm, out_hbm, idx_v, pos_v, loc_v, rows_v,
          chunk_sh):
    cid = lax.axis_index("c")
    sid = lax.axis_index("s")

    pltpu.sync_copy(idx_hbm.at[sid], idx_v)

    lane = lax.iota(jnp.int32, 16)

    def per_chunk(k, _):
        c = cid * CPC + k
        lo = c * MC

        row_off = jnp.minimum(sid * ROWS_PER_TILE, LAST_OFF)
        pltpu.sync_copy(
            mem_hbm.at[pl.ds(lo + row_off, ROWS_PER_TILE)],
            chunk_sh.at[pl.ds(row_off, ROWS_PER_TILE)],
        )
        plsc.subcore_barrier()

        def scan(j, off):
            v = idx_v[pl.ds(j * 16, 16)]
            m = (v >= lo) & (v < lo + MC)
            posvec = (sid * SLICE + j * 16) + lane
            tgt = (off - 1) + plsc.cumsum(m.astype(jnp.int32))
            plsc.store_scatter(pos_v, [tgt], posvec, mask=m)
            plsc.store_scatter(loc_v, [tgt], v - lo, mask=m)
            return off + jnp.sum(m.astype(jnp.int32))

        off = lax.fori_loop(0, NVEC, scan, jnp.int32(0))

        pos_v[pl.ds(off, 16)] = (sid * SLICE) + lane
        loc_v[pl.ds(off, 16)] = MC + lane

        nb = (off + 15) // 16

        def drain(b, _):
            pv = pos_v[pl.ds(b * 16, 16)]
            lv = loc_v[pl.ds(b * 16, 16)]
            pltpu.sync_copy(val_hbm.at[pv], rows_v)
            pltpu.sync_copy(rows_v, chunk_sh.at[lv], add=True)
            return 0

        lax.fori_loop(0, nb, drain, 0)
        plsc.subcore_barrier()

        pltpu.sync_copy(
            chunk_sh.at[pl.ds(row_off, ROWS_PER_TILE)],
            out_hbm.at[pl.ds(lo + row_off, ROWS_PER_TILE)],
        )
        return 0

    lax.fori_loop(0, CPC, per_chunk, 0)


@jax.jit
def _scatter_add(mem, idx, val):
    mesh = plsc.VectorSubcoreMesh(core_axis_name="c", subcore_axis_name="s")
    return pl.kernel(
        _body,
        mesh=mesh,
        compiler_params=pltpu.CompilerParams(
            use_tc_tiling_on_sc=False, needs_layout_passes=False
        ),
        out_type=jax.ShapeDtypeStruct((M, D), jnp.float32),
        scratch_types=[
            pltpu.VMEM((SLICE,), jnp.int32),
            pltpu.VMEM((PCAP,), jnp.int32),
            pltpu.VMEM((PCAP,), jnp.int32),
            pltpu.VMEM((16, D), jnp.float32),
            pltpu.VMEM_SHARED((MC + TRASH, D), jnp.float32),
        ],
    )(mem, idx.reshape(NS, SLICE), val)


def kernel(mem, idx, val):
    return _scatter_add(mem, idx, val)


# speedup vs baseline: 1.3194x; 1.3194x over previous
import functools

import jax
import jax.numpy as jnp
from jax import lax
from jax.experimental import pallas as pl
from jax.experimental.pallas import tpu as pltpu
from jax.experimental.pallas import tpu_sc as plsc

M = 1000000
D = 32
B = 819200

NC = 2
NS = 16
MC = 25000
NCHUNK = M // MC
CPC = NCHUNK // NC
SLICE = B // NS
NVEC = SLICE // 16
ROWS_PER_TILE = 1568
LAST_OFF = MC - ROWS_PER_TILE
TRASH = 16
PCAP = 2816
GRP = 8


def _body(mem_hbm, idx_hbm, val_hbm, out_hbm, idx_v, pos_v, loc_v, rows_v,
          chunk_sh, gsem, ssem):
    cid = lax.axis_index("c")
    sid = lax.axis_index("s")

    pltpu.sync_copy(idx_hbm.at[sid], idx_v)

    lane = lax.iota(jnp.int32, 16)

    def per_chunk(k, _):
        c = cid * CPC + k
        lo = c * MC

        row_off = jnp.minimum(sid * ROWS_PER_TILE, LAST_OFF)
        pltpu.sync_copy(
            mem_hbm.at[pl.ds(lo + row_off, ROWS_PER_TILE)],
            chunk_sh.at[pl.ds(row_off, ROWS_PER_TILE)],
        )
        plsc.subcore_barrier()

        def scan(j, off):
            v = idx_v[pl.ds(j * 16, 16)]
            m = (v >= lo) & (v < lo + MC)
            posvec = (sid * SLICE + j * 16) + lane
            tgt = (off - 1) + plsc.cumsum(m.astype(jnp.int32))
            plsc.store_scatter(pos_v, [tgt], posvec, mask=m)
            plsc.store_scatter(loc_v, [tgt], v - lo, mask=m)
            return off + jnp.sum(m.astype(jnp.int32))

        off = lax.fori_loop(0, NVEC, scan, jnp.int32(0))

        for r in range(GRP):
            pos_v[pl.ds(off + r * 16, 16)] = (sid * SLICE + r * 16) + lane
            loc_v[pl.ds(off + r * 16, 16)] = MC + lane

        ng = (off + GRP * 16 - 1) // (GRP * 16)

        def drain(g, _):
            base = g * GRP
            gathers = []
            for r in range(GRP):
                pv = pos_v[pl.ds((base + r) * 16, 16)]
                gathers.append(
                    pltpu.async_copy(val_hbm.at[pv], rows_v.at[r], gsem))
            for h in gathers:
                h.wait()
            adds = []
            for r in range(GRP):
                lv = loc_v[pl.ds((base + r) * 16, 16)]
                adds.append(
                    pltpu.async_copy(rows_v.at[r], chunk_sh.at[lv], ssem,
                                     add=True))
            for h in adds:
                h.wait()
            return 0

        lax.fori_loop(0, ng, drain, 0)
        plsc.subcore_barrier()

        pltpu.sync_copy(
            chunk_sh.at[pl.ds(row_off, ROWS_PER_TILE)],
            out_hbm.at[pl.ds(lo + row_off, ROWS_PER_TILE)],
        )
        return 0

    lax.fori_loop(0, CPC, per_chunk, 0)


@jax.jit
def _scatter_add(mem, idx, val):
    mesh = plsc.VectorSubcoreMesh(core_axis_name="c", subcore_axis_name="s")
    return pl.kernel(
        _body,
        mesh=mesh,
        compiler_params=pltpu.CompilerParams(
            use_tc_tiling_on_sc=False, needs_layout_passes=False
        ),
        out_type=jax.ShapeDtypeStruct((M, D), jnp.float32),
        scratch_types=[
            pltpu.VMEM((SLICE,), jnp.int32),
            pltpu.VMEM((PCAP,), jnp.int32),
            pltpu.VMEM((PCAP,), jnp.int32),
            pltpu.VMEM((GRP, 16, D), jnp.float32),
            pltpu.VMEM_SHARED((MC + TRASH, D), jnp.float32),
            pltpu.SemaphoreType.DMA,
            pltpu.SemaphoreType.DMA,
        ],
    )(mem, idx.reshape(NS, SLICE), val)


def kernel(mem, idx, val):
    return _scatter_add(mem, idx, val)
